# Initial kernel scaffold; baseline (speedup 1.0000x reference)
import jax
import jax.numpy as jnp
from jax.experimental import pallas as pl


def kernel(x, gcn_weight, prompt_w, edge_index, idx, labels):
    raise NotImplementedError("write your pallas kernel here")


# speedup vs baseline: 3.8761x; 3.8761x over previous
import functools

import jax
import jax.numpy as jnp
from jax import lax
from jax.experimental import pallas as pl
from jax.experimental.pallas import tpu as pltpu
from jax.experimental.pallas import tpu_sc as plsc

N = 10000
E = 320000
D = 128
NB = 10
NIDX = 2048

NC = 2
NS = 16
NW = NC * NS

NPAD = 10240
CH = 128
CPT = 157
EPAD = NS * CPT * CH
EDGES_PER_TILE = CPT * CH
RPT = NPAD // NS
KI = NIDX // NW

_mesh = plsc.VectorSubcoreMesh(core_axis_name="c", subcore_axis_name="s")


@functools.partial(
    pl.kernel,
    out_type=jax.ShapeDtypeStruct((2 * NPAD, D), jnp.float32),
    mesh=_mesh,
    scratch_types=[
        pltpu.VMEM((CH,), jnp.int32),
        pltpu.VMEM((CH,), jnp.int32),
        pltpu.VMEM((CH, D), jnp.float32),
        pltpu.VMEM((CH, D), jnp.float32),
        pltpu.VMEM_SHARED((NPAD, D), jnp.float32),
        pltpu.SemaphoreType.DMA,
    ],
)
def _seg_sum(x_hbm, src_hbm, dst_hbm, ones_hbm, zrow_hbm, out,
             src_v, dst_v, rows_v, ones_v, acc_sh, sem):
    c = lax.axis_index("c")
    s = lax.axis_index("s")

    pltpu.sync_copy(zrow_hbm, rows_v)
    for k in range(RPT // CH):
        pltpu.sync_copy(rows_v, acc_sh.at[pl.ds(s * RPT + k * CH, CH)])
    pltpu.sync_copy(ones_hbm, ones_v)
    plsc.subcore_barrier()

    base = s * EDGES_PER_TILE

    @pl.when(c == 0)
    def _():
        @pl.loop(0, CPT)
        def _(t):
            off = base + t * CH
            pltpu.sync_copy(src_hbm.at[pl.ds(off, CH)], src_v)
            pltpu.sync_copy(dst_hbm.at[pl.ds(off, CH)], dst_v)
            pltpu.async_copy(x_hbm.at[src_v], rows_v, sem).wait()
            pltpu.sync_copy(rows_v, acc_sh.at[dst_v], add=True)

    @pl.when(c == 1)
    def _():
        @pl.loop(0, CPT)
        def _(t):
            off = base + t * CH
            pltpu.sync_copy(dst_hbm.at[pl.ds(off, CH)], dst_v)
            pltpu.sync_copy(ones_v, acc_sh.at[dst_v], add=True)

    plsc.subcore_barrier()

    for k in range(RPT // CH):
        r = s * RPT + k * CH
        pltpu.sync_copy(acc_sh.at[pl.ds(r, CH)], rows_v)
        pltpu.sync_copy(rows_v, out.at[pl.ds(c * NPAD + r, CH)])


@functools.partial(
    pl.kernel,
    out_type=(
        jax.ShapeDtypeStruct((NIDX, D), jnp.float32),
        jax.ShapeDtypeStruct((NIDX, D), jnp.float32),
    ),
    mesh=_mesh,
    scratch_types=[
        pltpu.VMEM((KI,), jnp.int32),
        pltpu.VMEM((KI,), jnp.int32),
        pltpu.VMEM((KI, D), jnp.float32),
        pltpu.SemaphoreType.DMA,
    ],
)
def _gather_rows(tbl, idx_hbm, idxn_hbm, ga, gc, idx_v, idxn_v, rows_v, sem):
    c = lax.axis_index("c")
    s = lax.axis_index("s")
    wid = c * NS + s
    base = wid * KI
    pltpu.sync_copy(idx_hbm.at[pl.ds(base, KI)], idx_v)
    pltpu.sync_copy(idxn_hbm.at[pl.ds(base, KI)], idxn_v)
    pltpu.async_copy(tbl.at[idx_v], rows_v, sem).wait()
    pltpu.sync_copy(rows_v, ga.at[pl.ds(base, KI)])
    pltpu.async_copy(tbl.at[idxn_v], rows_v, sem).wait()
    pltpu.sync_copy(rows_v, gc.at[pl.ds(base, KI)])


def _dense_body(ga, gc, w_ref, pw_ref, lab_ref, out_ref):
    agg = ga[...]
    cnt = gc[...][:, 0:1]
    h = agg / jnp.maximum(cnt, 1.0)
    e = jax.lax.dot_general(h, w_ref[...], (((1,), (0,)), ((), ())),
                            preferred_element_type=jnp.float32)
    e = jnp.maximum(e, 0.0) * pw_ref[...]
    lab = lab_ref[...]
    cid = jax.lax.broadcasted_iota(jnp.int32, (NIDX, D), 1)
    oh = (lab == cid).astype(jnp.float32)
    sums = jax.lax.dot_general(oh, e, (((0,), (0,)), ((), ())),
                               preferred_element_type=jnp.float32)
    ones_col = jnp.full((NIDX, 1), 1.0, jnp.float32)
    counts = jax.lax.dot_general(oh, ones_col, (((0,), (0,)), ((), ())),
                                 preferred_element_type=jnp.float32)
    ave = sums / jnp.maximum(counts, 1.0)
    rn = e / jnp.maximum(jnp.sqrt(jnp.sum(e * e, axis=1, keepdims=True)), 1e-12)
    an = ave / jnp.maximum(jnp.sqrt(jnp.sum(ave * ave, axis=1, keepdims=True)), 1e-12)
    logits = jax.lax.dot_general(rn, an, (((1,), (1,)), ((), ())),
                                 preferred_element_type=jnp.float32)
    out_ref[...] = logits[:, :NB]


_dense = pl.pallas_call(
    _dense_body,
    out_shape=jax.ShapeDtypeStruct((NIDX, NB), jnp.float32),
)


def kernel(x, gcn_weight, prompt_w, edge_index, idx, labels):
    src = edge_index[0]
    dst = edge_index[1]
    pad = EPAD - E
    src_p = jnp.concatenate([src, jnp.zeros((pad,), jnp.int32)])
    dst_p = jnp.concatenate([dst, jnp.full((pad,), N, jnp.int32)])
    ones_rows = jnp.ones((CH, D), jnp.float32)
    zrow = jnp.zeros((CH, D), jnp.float32)
    tbl = _seg_sum(x, src_p, dst_p, ones_rows, zrow)
    idxn = idx + NPAD
    ga, gc = _gather_rows(tbl, idx, idxn)
    labels2d = labels.reshape(NIDX, 1)
    return _dense(ga, gc, gcn_weight, prompt_w, labels2d)


# speedup vs baseline: 11.8998x; 3.0701x over previous
import functools

import jax
import jax.numpy as jnp
from jax import lax
from jax.experimental import pallas as pl
from jax.experimental.pallas import tpu as pltpu
from jax.experimental.pallas import tpu_sc as plsc

N = 10000
E = 320000
D = 128
NB = 10
NIDX = 2048

NC = 2
NS = 16
NW = NC * NS

NPAD = 10240
CH = 128
CPT = 79
EPT = CPT * CH
EPAD = NW * EPT
DUMP = NIDX
TBL = NIDX + CH
PB = 10240
NBMAX = PB // CH
KI = NIDX // NS

_mesh = plsc.VectorSubcoreMesh(core_axis_name="c", subcore_axis_name="s")
_CP = pltpu.CompilerParams(needs_layout_passes=False)


@functools.partial(
    pl.kernel,
    out_type=(
        jax.ShapeDtypeStruct((2 * NIDX, D), jnp.float32),
        jax.ShapeDtypeStruct((2 * NIDX, D), jnp.float32),
    ),
    mesh=_mesh,
    compiler_params=_CP,
    scratch_types=[
        pltpu.VMEM((NPAD,), jnp.int32),
        pltpu.VMEM((NIDX,), jnp.int32),
        pltpu.VMEM((EPT,), jnp.int32),
        pltpu.VMEM((EPT,), jnp.int32),
        pltpu.VMEM((PB,), jnp.int32),
        pltpu.VMEM((PB,), jnp.int32),
        pltpu.VMEM((NBMAX, CH), jnp.int32),
        pltpu.VMEM((CH, D), jnp.float32),
        pltpu.VMEM((CH, D), jnp.float32),
        pltpu.VMEM((KI,), jnp.int32),
        pltpu.VMEM_SHARED((TBL, D), jnp.float32),
        pltpu.VMEM_SHARED((TBL, D), jnp.float32),
        pltpu.SemaphoreType.DMA,
    ],
)
def _seg_pool(x_hbm, src_hbm, dst_hbm, idx_hbm, rinit_hbm, zsrc_hbm,
              ones_hbm, zrow_hbm,
              ga, gc,
              rmp_v, idx_v, srcs_v, dsts_v, csrc_v, cslot_v, cslot2d,
              rows_v, ones_v, gidx_v, acc_sh, cnt_sh, sem):
    c = lax.axis_index("c")
    s = lax.axis_index("s")
    wid = c * NS + s

    pltpu.sync_copy(rinit_hbm, rmp_v)
    pltpu.sync_copy(idx_hbm, idx_v)

    @pl.loop(0, NIDX // 16)
    def _(j):
        ivec = idx_v[pl.ds(j * 16, 16)]
        svec = j * 16 + lax.broadcasted_iota(jnp.int32, (16,), 0)
        plsc.store_scatter(rmp_v, [ivec], svec)

    pltpu.sync_copy(zrow_hbm, rows_v)
    r0 = s * (TBL // NS)
    pltpu.sync_copy(rows_v, acc_sh.at[pl.ds(r0, CH)])
    pltpu.sync_copy(rows_v.at[pl.ds(0, TBL // NS - CH)],
                    acc_sh.at[pl.ds(r0 + CH, TBL // NS - CH)])
    pltpu.sync_copy(rows_v, cnt_sh.at[pl.ds(r0, CH)])
    pltpu.sync_copy(rows_v.at[pl.ds(0, TBL // NS - CH)],
                    cnt_sh.at[pl.ds(r0 + CH, TBL // NS - CH)])
    pltpu.sync_copy(ones_hbm, ones_v)

    pltpu.sync_copy(src_hbm.at[pl.ds(wid * EPT, EPT)], srcs_v)
    pltpu.sync_copy(dst_hbm.at[pl.ds(wid * EPT, EPT)], dsts_v)
    pltpu.sync_copy(zsrc_hbm, csrc_v)
    pltpu.sync_copy(rinit_hbm.at[pl.ds(0, PB)], cslot_v)
    plsc.subcore_barrier()

    def fbody(j, wp):
        dvec = dsts_v[pl.ds(j * 16, 16)]
        svec = srcs_v[pl.ds(j * 16, 16)]
        rvec = plsc.load_gather(rmp_v, [dvec])
        m = rvec < DUMP
        plsc.store_compressed(cslot_v.at[pl.ds(wp, 16)], rvec, mask=m)
        plsc.store_compressed(csrc_v.at[pl.ds(wp, 16)], svec, mask=m)
        return wp + plsc.all_reduce_population_count(m)[0]

    wp = lax.fori_loop(0, EPT // 16, fbody, 0)
    nb = (wp + CH - 1) // CH

    @pl.loop(0, nb * (CH // 16))
    def _(k):
        row = k // (CH // 16)
        lane = (k % (CH // 16)) * 16
        cslot2d[row, pl.ds(lane, 16)] = cslot_v[pl.ds(k * 16, 16)]

    @pl.loop(0, nb)
    def _(t):
        pltpu.async_copy(x_hbm.at[csrc_v.at[pl.ds(t * CH, CH)]], rows_v, sem).wait()
        pltpu.sync_copy(rows_v, acc_sh.at[cslot2d.at[t]], add=True)
        pltpu.sync_copy(ones_v, cnt_sh.at[cslot2d.at[t]], add=True)

    plsc.subcore_barrier()

    @pl.loop(0, KI // 16)
    def _(j):
        ivec = idx_v[pl.ds(s * KI + j * 16, 16)]
        slotv = plsc.load_gather(rmp_v, [ivec])
        gidx_v[pl.ds(j * 16, 16)] = slotv

    base = c * NIDX + s * KI
    pltpu.async_copy(acc_sh.at[gidx_v], rows_v, sem).wait()
    pltpu.sync_copy(rows_v, ga.at[pl.ds(base, KI)])
    pltpu.async_copy(cnt_sh.at[gidx_v], rows_v, sem).wait()
    pltpu.sync_copy(rows_v, gc.at[pl.ds(base, KI)])


def _dense_body(ga, gc, w_ref, pw_ref, lab_ref, out_ref):
    agg = ga[0:NIDX, :] + ga[NIDX:2 * NIDX, :]
    cnt = (gc[0:NIDX, :] + gc[NIDX:2 * NIDX, :])[:, 0:1]
    h = agg / jnp.maximum(cnt, 1.0)
    e = jax.lax.dot_general(h, w_ref[...], (((1,), (0,)), ((), ())),
                            preferred_element_type=jnp.float32)
    e = jnp.maximum(e, 0.0) * pw_ref[...]
    lab = lab_ref[...]
    cid = jax.lax.broadcasted_iota(jnp.int32, (NIDX, D), 1)
    oh = (lab == cid).astype(jnp.float32)
    sums = jax.lax.dot_general(oh, e, (((0,), (0,)), ((), ())),
                               preferred_element_type=jnp.float32)
    ones_col = jnp.full((NIDX, 1), 1.0, jnp.float32)
    counts = jax.lax.dot_general(oh, ones_col, (((0,), (0,)), ((), ())),
                                 preferred_element_type=jnp.float32)
    ave = sums / jnp.maximum(counts, 1.0)
    rn = e / jnp.maximum(jnp.sqrt(jnp.sum(e * e, axis=1, keepdims=True)), 1e-12)
    an = ave / jnp.maximum(jnp.sqrt(jnp.sum(ave * ave, axis=1, keepdims=True)), 1e-12)
    logits = jax.lax.dot_general(rn, an, (((1,), (1,)), ((), ())),
                                 preferred_element_type=jnp.float32)
    out_ref[...] = logits[:, :NB]


_dense = pl.pallas_call(
    _dense_body,
    out_shape=jax.ShapeDtypeStruct((NIDX, NB), jnp.float32),
)


def kernel(x, gcn_weight, prompt_w, edge_index, idx, labels):
    src = edge_index[0]
    dst = edge_index[1]
    pad = EPAD - E
    src_p = jnp.concatenate([src, jnp.zeros((pad,), jnp.int32)])
    dst_p = jnp.concatenate([dst, jnp.full((pad,), N, jnp.int32)])
    rinit = jnp.full((NPAD,), DUMP, jnp.int32)
    zsrc = jnp.zeros((PB,), jnp.int32)
    ones_rows = jnp.ones((CH, D), jnp.float32)
    zrow = jnp.zeros((CH, D), jnp.float32)
    ga, gc = _seg_pool(x, src_p, dst_p, idx, rinit, zsrc, ones_rows, zrow)
    labels2d = labels.reshape(NIDX, 1)
    return _dense(ga, gc, gcn_weight, prompt_w, labels2d)
